# baseline (speedup 1.0000x reference)
import dataclasses
import functools

import jax
import jax.numpy as jnp
from jax import lax
from jax.experimental import pallas as pl
from jax.experimental.pallas import tpu as pltpu
from jax.experimental.pallas import tpu_sc as plsc

B = 1024
FDIM = 128
NUM_CLS = 100000
SAMP = 8192
NNBR = 10
PAD_CLS = 100352
ROWS = PAD_CLS // 128
NBLK = 50
BK = NUM_CLS // NBLK
CHUNK = PAD_CLS // 32
NEG = float(jnp.finfo(jnp.float32).min)

@functools.lru_cache(maxsize=None)
def _vmesh():
    return plsc.VectorSubcoreMesh(core_axis_name="c", subcore_axis_name="s")


@functools.lru_cache(maxsize=None)
def _sc_params():
    cp = pltpu.CompilerParams()
    if "needs_layout_passes" in pltpu.CompilerParams.__dataclass_fields__:
        cp = dataclasses.replace(cp, needs_layout_passes=False)
    return cp


def _topk_body(feat_ref, w_ref, out_ref, fn_s, runv_s, runi_s, s_ref,
               flag_s):
    i = pl.program_id(0)

    @pl.when(i == 0)
    def _init():
        f = feat_ref[...]
        nrm = jnp.sqrt(jnp.sum(f * f, axis=1, keepdims=True)) + 1e-12
        fn_s[...] = f / nrm
        runv_s[...] = jnp.full((B, 16), NEG, jnp.float32)
        runi_s[...] = jnp.zeros((B, 16), jnp.int32)

    w = w_ref[...]
    wn = w / (jnp.sqrt(jnp.sum(w * w, axis=1, keepdims=True)) + 1e-12)
    s_ref[...] = lax.dot_general(fn_s[...], wn, (((1,), (1,)), ((), ())),
                                 preferred_element_type=jnp.float32)

    flag_s[0] = jnp.sum(
        (s_ref[...] > runv_s[:, 9:10]).astype(jnp.int32))

    for _ in range(NNBR):
        @pl.when(flag_s[0] > 0)
        def _extract():
            sv = s_ref[...]
            thr = runv_s[:, 9:10]
            colio = lax.broadcasted_iota(jnp.int32, (B, BK), 1)
            lane16 = lax.broadcasted_iota(jnp.int32, (B, 16), 1)
            m = jnp.max(sv, axis=1, keepdims=True)
            idx = jnp.min(jnp.where(sv == m, colio, BK), axis=1,
                          keepdims=True)
            do = m > thr
            sv2 = jnp.where(jnp.logical_and(do, colio == idx), NEG, sv)
            s_ref[...] = sv2
            gid = idx + i * BK
            rv = runv_s[...]
            ri = runi_s[...]
            pos = jnp.sum((rv >= m).astype(jnp.int32), axis=1, keepdims=True)
            rv_shift = jnp.concatenate(
                [jnp.full((B, 1), NEG, jnp.float32), rv[:, :15]], axis=1)
            ri_shift = jnp.concatenate(
                [jnp.zeros((B, 1), jnp.int32), ri[:, :15]], axis=1)
            nrv = jnp.where(lane16 < pos, rv,
                            jnp.where(lane16 == pos, m, rv_shift))
            nri = jnp.where(lane16 < pos, ri,
                            jnp.where(lane16 == pos, gid, ri_shift))
            nrv = jnp.where(do, nrv, rv)
            runi_s[...] = jnp.where(do, nri, ri)
            runv_s[...] = nrv
            flag_s[0] = jnp.sum((sv2 > nrv[:, 9:10]).astype(jnp.int32))

    @pl.when(i == NBLK - 1)
    def _emit():
        out_ref[...] = jnp.concatenate(
            [runi_s[...], jnp.zeros((B, 112), jnp.int32)], axis=1)


def _run_topk(features, W):
    return pl.pallas_call(
        _topk_body,
        grid=(NBLK,),
        in_specs=[
            pl.BlockSpec((B, FDIM), lambda i: (0, 0)),
            pl.BlockSpec((BK, FDIM), lambda i: (i, 0)),
        ],
        out_specs=pl.BlockSpec((B, 128), lambda i: (0, 0)),
        out_shape=jax.ShapeDtypeStruct((B, 128), jnp.int32),
        scratch_shapes=[
            pltpu.VMEM((B, FDIM), jnp.float32),
            pltpu.VMEM((B, 16), jnp.float32),
            pltpu.VMEM((B, 16), jnp.int32),
            pltpu.VMEM((B, BK), jnp.float32),
            pltpu.SMEM((1,), jnp.int32),
        ],
    )(features, W)


@functools.lru_cache(maxsize=None)
def _scatter_masks_kernel():
    return functools.partial(
        pl.kernel,
        mesh=_vmesh(),
        out_type=(jax.ShapeDtypeStruct((PAD_CLS,), jnp.float32),
                  jax.ShapeDtypeStruct((PAD_CLS,), jnp.float32)),
        scratch_types=[pltpu.VMEM((PAD_CLS,), jnp.float32),
                       pltpu.VMEM((B * NNBR,), jnp.int32)],
        compiler_params=_sc_params(),
    )(_scatter_masks_body)


def _scatter_masks_body(nbr_hbm, lab_hbm, ma_hbm, mb_hbm, mask_v, idx_v):
    cid = lax.axis_index("c")
    sid = lax.axis_index("s")
    zeros16 = jnp.zeros((16,), jnp.float32)
    ones16 = jnp.ones((16,), jnp.float32)

    @pl.when(jnp.logical_and(cid == 0, sid == 0))
    def _nbr_mask():
        @pl.loop(0, PAD_CLS, step=16)
        def _(j):
            mask_v[pl.ds(j, 16)] = zeros16

        pltpu.sync_copy(nbr_hbm, idx_v)

        @pl.loop(0, B * NNBR, step=16)
        def _(j):
            plsc.store_scatter(mask_v, [idx_v[pl.ds(j, 16)]], ones16)

        pltpu.sync_copy(mask_v, ma_hbm)

    @pl.when(jnp.logical_and(cid == 1, sid == 0))
    def _lab_mask():
        @pl.loop(0, PAD_CLS, step=16)
        def _(j):
            mask_v[pl.ds(j, 16)] = zeros16

        pltpu.sync_copy(lab_hbm, idx_v.at[pl.ds(0, B)])

        @pl.loop(0, B, step=16)
        def _(j):
            plsc.store_scatter(mask_v, [idx_v[pl.ds(j, 16)]], ones16)

        pltpu.sync_copy(mask_v, mb_hbm)


def _positions_body(ma_ref, mb_ref, pos_ref, val_ref, c2x_ref):
    m2 = mb_ref[...]
    m1 = ma_ref[...] * (1.0 - m2)
    r = lax.broadcasted_iota(jnp.int32, (128, 128), 0)
    c = lax.broadcasted_iota(jnp.int32, (128, 128), 1)
    upper = (r < c).astype(jnp.float32)
    rr = lax.broadcasted_iota(jnp.int32, (ROWS, ROWS), 0)
    cc = lax.broadcasted_iota(jnp.int32, (ROWS, ROWS), 1)
    lower = (cc < rr).astype(jnp.float32)

    def xcum(m):
        pre = lax.dot_general(m, upper, (((1,), (0,)), ((), ())),
                              preferred_element_type=jnp.float32)
        rs = jnp.sum(m, axis=1, keepdims=True)
        off = lax.dot_general(lower, rs, (((1,), (0,)), ((), ())),
                              preferred_element_type=jnp.float32)
        return pre + off

    c2 = xcum(m2)
    c1 = xcum(m1)
    n2 = jnp.sum(m2)
    n1 = jnp.sum(m1)
    ii = (lax.broadcasted_iota(jnp.int32, (ROWS, 128), 0) * 128
          + lax.broadcasted_iota(jnp.int32, (ROWS, 128), 1)).astype(jnp.float32)
    pos = jnp.where(m2 > 0.5, c2,
                    jnp.where(m1 > 0.5, n2 + c1, n2 + n1 + (ii - c2 - c1)))
    valid = jnp.logical_and(ii < float(NUM_CLS), pos < float(SAMP))
    pos_ref[...] = pos.astype(jnp.int32)
    val_ref[...] = valid.astype(jnp.int32)
    c2x_ref[...] = c2.astype(jnp.int32)


def _run_positions(maskA, maskB):
    return pl.pallas_call(
        _positions_body,
        out_shape=(jax.ShapeDtypeStruct((ROWS, 128), jnp.int32),
                   jax.ShapeDtypeStruct((ROWS, 128), jnp.int32),
                   jax.ShapeDtypeStruct((ROWS, 128), jnp.int32)),
    )(maskA.reshape(ROWS, 128), maskB.reshape(ROWS, 128))


@functools.lru_cache(maxsize=None)
def _compact_and_ranks_kernel():
    return functools.partial(
        pl.kernel,
        mesh=_vmesh(),
        out_type=(jax.ShapeDtypeStruct((SAMP,), jnp.int32),
                  jax.ShapeDtypeStruct((B,), jnp.int32)),
        scratch_types=[pltpu.VMEM((SAMP + 16,), jnp.int32),
                       pltpu.VMEM((CHUNK,), jnp.int32),
                       pltpu.VMEM((CHUNK,), jnp.int32),
                       pltpu.VMEM((PAD_CLS,), jnp.int32),
                       pltpu.VMEM((B,), jnp.int32),
                       pltpu.VMEM((B,), jnp.int32)],
        compiler_params=_sc_params(),
    )(_compact_and_ranks_body)


def _compact_and_ranks_body(pos_hbm, val_hbm, c2x_hbm, lab_hbm, sel_hbm,
                            idxs_hbm, sel_v, chp_v, chv_v, c2x_v, lab_v,
                            out_v):
    cid = lax.axis_index("c")
    sid = lax.axis_index("s")

    @pl.when(jnp.logical_and(cid == 0, sid == 0))
    def _compact():
        @pl.loop(0, SAMP + 16, step=16)
        def _(j):
            sel_v[pl.ds(j, 16)] = jnp.zeros((16,), jnp.int32)

        @pl.loop(0, 32)
        def _(ch):
            pltpu.sync_copy(pos_hbm.at[pl.ds(ch * CHUNK, CHUNK)], chp_v)
            pltpu.sync_copy(val_hbm.at[pl.ds(ch * CHUNK, CHUNK)], chv_v)

            @pl.loop(0, CHUNK, step=16)
            def _(k):
                p = jnp.minimum(chp_v[pl.ds(k, 16)], SAMP)
                ok = chv_v[pl.ds(k, 16)] > 0
                gid = (ch * CHUNK + k
                       + lax.broadcasted_iota(jnp.int32, (16,), 0))
                plsc.store_scatter(sel_v, [p], gid, mask=ok)

        pltpu.sync_copy(sel_v.at[pl.ds(0, SAMP)], sel_hbm)

    @pl.when(jnp.logical_and(cid == 1, sid == 0))
    def _ranks():
        pltpu.sync_copy(c2x_hbm, c2x_v)
        pltpu.sync_copy(lab_hbm, lab_v)

        @pl.loop(0, B, step=16)
        def _(k):
            out_v[pl.ds(k, 16)] = plsc.load_gather(
                c2x_v, [lab_v[pl.ds(k, 16)]])

        pltpu.sync_copy(out_v, idxs_hbm)


@functools.lru_cache(maxsize=None)
def _gather_rows_kernel():
    return functools.partial(
        pl.kernel,
        mesh=_vmesh(),
        out_type=jax.ShapeDtypeStruct((SAMP, FDIM), jnp.float32),
        scratch_types=[pltpu.VMEM((SAMP // 32,), jnp.int32),
                       pltpu.VMEM((SAMP // 32, FDIM), jnp.float32),
                       pltpu.SemaphoreType.DMA],
    )(_gather_rows_body)


def _gather_rows_body(sel_hbm, w_hbm, out_hbm, idx_v, rows_v, sem):
    wid = lax.axis_index("s") * 2 + lax.axis_index("c")
    base = wid * (SAMP // 32)
    pltpu.sync_copy(sel_hbm.at[pl.ds(base, SAMP // 32)], idx_v)
    pltpu.async_copy(w_hbm.at[idx_v], rows_v, sem).wait()
    pltpu.sync_copy(rows_v, out_hbm.at[pl.ds(base, SAMP // 32)])


def kernel(features, labels, W):
    nbr_pad = _run_topk(features, W)
    nbrs = nbr_pad[:, :NNBR].reshape(-1)
    maskA, maskB = _scatter_masks_kernel()(nbrs, labels)
    pos, valid, c2x = _run_positions(maskA, maskB)
    sel, idxs = _compact_and_ranks_kernel()(pos.reshape(-1),
                                            valid.reshape(-1),
                                            c2x.reshape(-1), labels)
    weights = _gather_rows_kernel()(sel, W)
    bias = jnp.zeros((SAMP,), jnp.float32)
    return weights, bias, idxs


# speedup vs baseline: 1.5535x; 1.5535x over previous
import dataclasses
import functools

import jax
import jax.numpy as jnp
from jax import lax
from jax.experimental import pallas as pl
from jax.experimental.pallas import tpu as pltpu
from jax.experimental.pallas import tpu_sc as plsc

B = 1024
FDIM = 128
NUM_CLS = 100000
SAMP = 8192
NNBR = 10
PAD_CLS = 100352
ROWS = PAD_CLS // 128
NBLK = 50
BK = NUM_CLS // NBLK
CHUNK = PAD_CLS // 32
NEG = float(jnp.finfo(jnp.float32).min)

@functools.lru_cache(maxsize=None)
def _vmesh():
    return plsc.VectorSubcoreMesh(core_axis_name="c", subcore_axis_name="s")


@functools.lru_cache(maxsize=None)
def _sc_params():
    cp = pltpu.CompilerParams()
    if "needs_layout_passes" in pltpu.CompilerParams.__dataclass_fields__:
        cp = dataclasses.replace(cp, needs_layout_passes=False)
    return cp


NCORE = 2
IBLK = NBLK // NCORE


def _topk_body(feat_ref, w_ref, outv_ref, outi_ref, fn_s, runv_s, runi_s):
    o = pl.program_id(0)
    i = pl.program_id(1)

    @pl.when(i == 0)
    def _init():
        f = feat_ref[...]
        nrm = jnp.sqrt(jnp.sum(f * f, axis=1, keepdims=True)) + 1e-12
        fn_s[...] = f / nrm
        runv_s[...] = jnp.full((B, 16), NEG, jnp.float32)
        runi_s[...] = jnp.zeros((B, 16), jnp.int32)

    w = w_ref[...]
    wn = w / (jnp.sqrt(jnp.sum(w * w, axis=1, keepdims=True)) + 1e-12)
    s = lax.dot_general(fn_s[...], wn, (((1,), (1,)), ((), ())),
                        preferred_element_type=jnp.float32)

    colio = lax.broadcasted_iota(jnp.int32, (B, BK), 1)
    lane16 = lax.broadcasted_iota(jnp.int32, (B, 16), 1)
    blk = o * IBLK + i
    for _ in range(NNBR):
        m = jnp.max(s, axis=1, keepdims=True)
        idx = jnp.min(jnp.where(s == m, colio, BK), axis=1, keepdims=True)
        s = jnp.where(colio == idx, NEG, s)
        gid = idx + blk * BK
        rv = runv_s[...]
        ri = runi_s[...]
        do = m > rv[:, 9:10]
        pos = jnp.sum((rv >= m).astype(jnp.int32), axis=1, keepdims=True)
        rv_shift = jnp.concatenate(
            [jnp.full((B, 1), NEG, jnp.float32), rv[:, :15]], axis=1)
        ri_shift = jnp.concatenate(
            [jnp.zeros((B, 1), jnp.int32), ri[:, :15]], axis=1)
        nrv = jnp.where(lane16 < pos, rv,
                        jnp.where(lane16 == pos, m, rv_shift))
        nri = jnp.where(lane16 < pos, ri,
                        jnp.where(lane16 == pos, gid, ri_shift))
        runv_s[...] = jnp.where(do, nrv, rv)
        runi_s[...] = jnp.where(do, nri, ri)

    @pl.when(i == IBLK - 1)
    def _emit():
        outv_ref[...] = runv_s[...].reshape(1, B, 16)
        outi_ref[...] = runi_s[...].reshape(1, B, 16)


def _run_topk(features, W):
    return pl.pallas_call(
        _topk_body,
        grid=(NCORE, IBLK),
        in_specs=[
            pl.BlockSpec((B, FDIM), lambda o, i: (0, 0)),
            pl.BlockSpec((BK, FDIM), lambda o, i: (o * IBLK + i, 0)),
        ],
        out_specs=[
            pl.BlockSpec((1, B, 16), lambda o, i: (o, 0, 0)),
            pl.BlockSpec((1, B, 16), lambda o, i: (o, 0, 0)),
        ],
        out_shape=[jax.ShapeDtypeStruct((NCORE, B, 16), jnp.float32),
                   jax.ShapeDtypeStruct((NCORE, B, 16), jnp.int32)],
        scratch_shapes=[
            pltpu.VMEM((B, FDIM), jnp.float32),
            pltpu.VMEM((B, 16), jnp.float32),
            pltpu.VMEM((B, 16), jnp.int32),
        ],
        compiler_params=pltpu.CompilerParams(
            dimension_semantics=("parallel", "arbitrary")),
    )(features, W)


def _merge_body(v_ref, i_ref, out_ref):
    catv = jnp.concatenate([v_ref[0], v_ref[1]], axis=1)
    cati = jnp.concatenate([i_ref[0], i_ref[1]], axis=1)
    cio = lax.broadcasted_iota(jnp.int32, (B, 32), 1)
    ni = []
    for _ in range(NNBR):
        m = jnp.max(catv, axis=1, keepdims=True)
        c = jnp.min(jnp.where(catv == m, cio, 32), axis=1, keepdims=True)
        hit = cio == c
        ni.append(jnp.sum(jnp.where(hit, cati, 0), axis=1, keepdims=True))
        catv = jnp.where(hit, NEG, catv)
    out_ref[...] = jnp.concatenate(
        ni + [jnp.zeros((B, 128 - NNBR), jnp.int32)], axis=1)


def _run_merge(vals, ids):
    return pl.pallas_call(
        _merge_body,
        out_shape=jax.ShapeDtypeStruct((B, 128), jnp.int32),
    )(vals, ids)


@functools.lru_cache(maxsize=None)
def _scatter_masks_kernel():
    return functools.partial(
        pl.kernel,
        mesh=_vmesh(),
        out_type=(jax.ShapeDtypeStruct((PAD_CLS,), jnp.float32),
                  jax.ShapeDtypeStruct((PAD_CLS,), jnp.float32)),
        scratch_types=[pltpu.VMEM((PAD_CLS,), jnp.float32),
                       pltpu.VMEM((B * NNBR,), jnp.int32)],
        compiler_params=_sc_params(),
    )(_scatter_masks_body)


def _scatter_masks_body(nbr_hbm, lab_hbm, ma_hbm, mb_hbm, mask_v, idx_v):
    cid = lax.axis_index("c")
    sid = lax.axis_index("s")
    zeros16 = jnp.zeros((16,), jnp.float32)
    ones16 = jnp.ones((16,), jnp.float32)

    @pl.when(jnp.logical_and(cid == 0, sid == 0))
    def _nbr_mask():
        @pl.loop(0, PAD_CLS, step=16)
        def _(j):
            mask_v[pl.ds(j, 16)] = zeros16

        pltpu.sync_copy(nbr_hbm, idx_v)

        @pl.loop(0, B * NNBR, step=16)
        def _(j):
            plsc.store_scatter(mask_v, [idx_v[pl.ds(j, 16)]], ones16)

        pltpu.sync_copy(mask_v, ma_hbm)

    @pl.when(jnp.logical_and(cid == 1, sid == 0))
    def _lab_mask():
        @pl.loop(0, PAD_CLS, step=16)
        def _(j):
            mask_v[pl.ds(j, 16)] = zeros16

        pltpu.sync_copy(lab_hbm, idx_v.at[pl.ds(0, B)])

        @pl.loop(0, B, step=16)
        def _(j):
            plsc.store_scatter(mask_v, [idx_v[pl.ds(j, 16)]], ones16)

        pltpu.sync_copy(mask_v, mb_hbm)


def _positions_body(ma_ref, mb_ref, pos_ref, val_ref, c2x_ref):
    m2 = mb_ref[...]
    m1 = ma_ref[...] * (1.0 - m2)
    r = lax.broadcasted_iota(jnp.int32, (128, 128), 0)
    c = lax.broadcasted_iota(jnp.int32, (128, 128), 1)
    upper = (r < c).astype(jnp.float32)
    rr = lax.broadcasted_iota(jnp.int32, (ROWS, ROWS), 0)
    cc = lax.broadcasted_iota(jnp.int32, (ROWS, ROWS), 1)
    lower = (cc < rr).astype(jnp.float32)

    def xcum(m):
        pre = lax.dot_general(m, upper, (((1,), (0,)), ((), ())),
                              preferred_element_type=jnp.float32)
        rs = jnp.sum(m, axis=1, keepdims=True)
        off = lax.dot_general(lower, rs, (((1,), (0,)), ((), ())),
                              preferred_element_type=jnp.float32)
        return pre + off

    c2 = xcum(m2)
    c1 = xcum(m1)
    n2 = jnp.sum(m2)
    n1 = jnp.sum(m1)
    ii = (lax.broadcasted_iota(jnp.int32, (ROWS, 128), 0) * 128
          + lax.broadcasted_iota(jnp.int32, (ROWS, 128), 1)).astype(jnp.float32)
    pos = jnp.where(m2 > 0.5, c2,
                    jnp.where(m1 > 0.5, n2 + c1, n2 + n1 + (ii - c2 - c1)))
    valid = jnp.logical_and(ii < float(NUM_CLS), pos < float(SAMP))
    pos_ref[...] = pos.astype(jnp.int32)
    val_ref[...] = valid.astype(jnp.int32)
    c2x_ref[...] = c2.astype(jnp.int32)


def _run_positions(maskA, maskB):
    return pl.pallas_call(
        _positions_body,
        out_shape=(jax.ShapeDtypeStruct((ROWS, 128), jnp.int32),
                   jax.ShapeDtypeStruct((ROWS, 128), jnp.int32),
                   jax.ShapeDtypeStruct((ROWS, 128), jnp.int32)),
    )(maskA.reshape(ROWS, 128), maskB.reshape(ROWS, 128))


@functools.lru_cache(maxsize=None)
def _compact_and_ranks_kernel():
    return functools.partial(
        pl.kernel,
        mesh=_vmesh(),
        out_type=(jax.ShapeDtypeStruct((SAMP,), jnp.int32),
                  jax.ShapeDtypeStruct((B,), jnp.int32)),
        scratch_types=[pltpu.VMEM((SAMP + 16,), jnp.int32),
                       pltpu.VMEM((CHUNK,), jnp.int32),
                       pltpu.VMEM((CHUNK,), jnp.int32),
                       pltpu.VMEM((PAD_CLS,), jnp.int32),
                       pltpu.VMEM((B,), jnp.int32),
                       pltpu.VMEM((B,), jnp.int32)],
        compiler_params=_sc_params(),
    )(_compact_and_ranks_body)


def _compact_and_ranks_body(pos_hbm, val_hbm, c2x_hbm, lab_hbm, sel_hbm,
                            idxs_hbm, sel_v, chp_v, chv_v, c2x_v, lab_v,
                            out_v):
    cid = lax.axis_index("c")
    sid = lax.axis_index("s")

    @pl.when(jnp.logical_and(cid == 0, sid == 0))
    def _compact():
        @pl.loop(0, SAMP + 16, step=16)
        def _(j):
            sel_v[pl.ds(j, 16)] = jnp.zeros((16,), jnp.int32)

        @pl.loop(0, 32)
        def _(ch):
            pltpu.sync_copy(pos_hbm.at[pl.ds(ch * CHUNK, CHUNK)], chp_v)
            pltpu.sync_copy(val_hbm.at[pl.ds(ch * CHUNK, CHUNK)], chv_v)

            @pl.loop(0, CHUNK, step=16)
            def _(k):
                p = jnp.minimum(chp_v[pl.ds(k, 16)], SAMP)
                ok = chv_v[pl.ds(k, 16)] > 0
                gid = (ch * CHUNK + k
                       + lax.broadcasted_iota(jnp.int32, (16,), 0))
                plsc.store_scatter(sel_v, [p], gid, mask=ok)

        pltpu.sync_copy(sel_v.at[pl.ds(0, SAMP)], sel_hbm)

    @pl.when(jnp.logical_and(cid == 1, sid == 0))
    def _ranks():
        pltpu.sync_copy(c2x_hbm, c2x_v)
        pltpu.sync_copy(lab_hbm, lab_v)

        @pl.loop(0, B, step=16)
        def _(k):
            out_v[pl.ds(k, 16)] = plsc.load_gather(
                c2x_v, [lab_v[pl.ds(k, 16)]])

        pltpu.sync_copy(out_v, idxs_hbm)


@functools.lru_cache(maxsize=None)
def _gather_rows_kernel():
    return functools.partial(
        pl.kernel,
        mesh=_vmesh(),
        out_type=jax.ShapeDtypeStruct((SAMP, FDIM), jnp.float32),
        scratch_types=[pltpu.VMEM((SAMP // 32,), jnp.int32),
                       pltpu.VMEM((SAMP // 32, FDIM), jnp.float32),
                       pltpu.SemaphoreType.DMA],
    )(_gather_rows_body)


def _gather_rows_body(sel_hbm, w_hbm, out_hbm, idx_v, rows_v, sem):
    wid = lax.axis_index("s") * 2 + lax.axis_index("c")
    base = wid * (SAMP // 32)
    pltpu.sync_copy(sel_hbm.at[pl.ds(base, SAMP // 32)], idx_v)
    pltpu.async_copy(w_hbm.at[idx_v], rows_v, sem).wait()
    pltpu.sync_copy(rows_v, out_hbm.at[pl.ds(base, SAMP // 32)])


def kernel(features, labels, W):
    vals, ids = _run_topk(features, W)
    nbr_pad = _run_merge(vals, ids)
    nbrs = nbr_pad[:, :NNBR].reshape(-1)
    maskA, maskB = _scatter_masks_kernel()(nbrs, labels)
    pos, valid, c2x = _run_positions(maskA, maskB)
    sel, idxs = _compact_and_ranks_kernel()(pos.reshape(-1),
                                            valid.reshape(-1),
                                            c2x.reshape(-1), labels)
    weights = _gather_rows_kernel()(sel, W)
    bias = jnp.zeros((SAMP,), jnp.float32)
    return weights, bias, idxs


# speedup vs baseline: 3.3040x; 2.1268x over previous
import dataclasses
import functools

import jax
import jax.numpy as jnp
from jax import lax
from jax.experimental import pallas as pl
from jax.experimental.pallas import tpu as pltpu
from jax.experimental.pallas import tpu_sc as plsc

B = 1024
FDIM = 128
NUM_CLS = 100000
SAMP = 8192
NNBR = 10
PAD_CLS = 100352
ROWS = PAD_CLS // 128
NBLK = 50
BK = NUM_CLS // NBLK
CHUNK = PAD_CLS // 32
NEG = float(jnp.finfo(jnp.float32).min)

@functools.lru_cache(maxsize=None)
def _vmesh():
    return plsc.VectorSubcoreMesh(core_axis_name="c", subcore_axis_name="s")


@functools.lru_cache(maxsize=None)
def _sc_params():
    cp = pltpu.CompilerParams()
    if "needs_layout_passes" in pltpu.CompilerParams.__dataclass_fields__:
        cp = dataclasses.replace(cp, needs_layout_passes=False)
    return cp


BKA = 2048
NBLKA = PAD_CLS // BKA
SEGS_PER_BLK = BKA // 128
NCAND = NNBR * 128


def _scores_body(feat_ref, w_ref, sc_ref, gm_ref, fn_s):
    i = pl.program_id(0)

    @pl.when(i == 0)
    def _init():
        f = feat_ref[...]
        nrm = jnp.sqrt(jnp.sum(f * f, axis=1, keepdims=True)) + 1e-12
        fn_s[...] = f / nrm

    w = w_ref[...]
    wn = w / (jnp.sqrt(jnp.sum(w * w, axis=1, keepdims=True)) + 1e-12)
    s = lax.dot_general(fn_s[...], wn, (((1,), (1,)), ((), ())),
                        preferred_element_type=jnp.float32)
    gcol = lax.broadcasted_iota(jnp.int32, (B, BKA), 1) + i * BKA
    s = jnp.where(gcol < NUM_CLS, s, NEG)
    sc_ref[...] = s
    gm_ref[...] = jnp.max(s.reshape(B, SEGS_PER_BLK, 128),
                          axis=2).reshape(1, B, SEGS_PER_BLK)


def _run_scores(features, Wp):
    return pl.pallas_call(
        _scores_body,
        grid=(NBLKA,),
        in_specs=[
            pl.BlockSpec((B, FDIM), lambda i: (0, 0)),
            pl.BlockSpec((BKA, FDIM), lambda i: (i, 0)),
        ],
        out_specs=[
            pl.BlockSpec((B, BKA), lambda i: (0, i)),
            pl.BlockSpec((1, B, SEGS_PER_BLK), lambda i: (i, 0, 0)),
        ],
        out_shape=[jax.ShapeDtypeStruct((B, PAD_CLS), jnp.float32),
                   jax.ShapeDtypeStruct((NBLKA, B, SEGS_PER_BLK),
                                        jnp.float32)],
        scratch_shapes=[pltpu.VMEM((B, FDIM), jnp.float32)],
    )(features, Wp)


def _segtop_body(gm_ref, flat_ref, seg_ref):
    gm = gm_ref[...]
    colio = lax.broadcasted_iota(jnp.int32, (B, ROWS), 1)
    segs = []
    for _ in range(NNBR):
        m = jnp.max(gm, axis=1, keepdims=True)
        sid = jnp.min(jnp.where(gm == m, colio, ROWS), axis=1, keepdims=True)
        gm = jnp.where(colio == sid, NEG, gm)
        segs.append(sid)
    seg = jnp.concatenate(segs + [jnp.zeros((B, 6), jnp.int32)], axis=1)
    seg_ref[...] = seg
    rowio = lax.broadcasted_iota(jnp.int32, (B, 16), 0)
    lane16 = lax.broadcasted_iota(jnp.int32, (B, 16), 1)
    flat_ref[...] = jnp.where(lane16 < NNBR, rowio * ROWS + seg, 0)


def _run_segtop(gm):
    return pl.pallas_call(
        _segtop_body,
        out_shape=[jax.ShapeDtypeStruct((B, 16), jnp.int32),
                   jax.ShapeDtypeStruct((B, 16), jnp.int32)],
    )(gm)


@functools.lru_cache(maxsize=None)
def _gather_cand_kernel():
    n_per = B * NNBR // 32

    @functools.partial(
        pl.kernel,
        mesh=_vmesh(),
        out_type=jax.ShapeDtypeStruct((B * NNBR, 128), jnp.float32),
        scratch_types=[pltpu.VMEM((n_per,), jnp.int32),
                       pltpu.VMEM((n_per, 128), jnp.float32),
                       pltpu.SemaphoreType.DMA],
    )
    def gather_cand(flat_hbm, table_hbm, out_hbm, idx_v, rows_v, sem):
        wid = lax.axis_index("s") * 2 + lax.axis_index("c")
        base = wid * n_per
        pltpu.sync_copy(flat_hbm.at[pl.ds(base, n_per)], idx_v)
        pltpu.async_copy(table_hbm.at[idx_v], rows_v, sem).wait()
        pltpu.sync_copy(rows_v, out_hbm.at[pl.ds(base, n_per)])

    return gather_cand


def _cand_body(cand_ref, seg_ref, gm_ref, ids_ref, flag_ref):
    s = cand_ref[...]
    colio = lax.broadcasted_iota(jnp.int32, (B, NCAND), 1)
    seg = seg_ref[...]
    l16 = lax.broadcasted_iota(jnp.int32, (B, 16), 1)
    ids = []
    lastm = None
    for _ in range(NNBR):
        m = jnp.max(s, axis=1, keepdims=True)
        c = jnp.min(jnp.where(s == m, colio, NCAND), axis=1, keepdims=True)
        s = jnp.where(colio == c, NEG, s)
        r = c // 128
        w = c - r * 128
        segsel = jnp.sum(jnp.where(l16 == r, seg, 0), axis=1, keepdims=True)
        ids.append(segsel * 128 + w)
        lastm = m
    ids_ref[...] = jnp.concatenate(
        ids + [jnp.zeros((B, 128 - NNBR), jnp.int32)], axis=1)
    gm = gm_ref[...]
    cnt = jnp.sum((gm >= lastm).astype(jnp.int32), axis=1)
    flag_ref[...] = jnp.reshape(jnp.sum((cnt > NNBR).astype(jnp.int32)),
                                (1, 1))


def _run_cand(cand, seg, gm):
    return pl.pallas_call(
        _cand_body,
        out_shape=[jax.ShapeDtypeStruct((B, 128), jnp.int32),
                   jax.ShapeDtypeStruct((1, 1), jnp.int32)],
    )(cand, seg, gm)


NCORE = 2
IBLK = NBLK // NCORE


def _topk_body(feat_ref, w_ref, outv_ref, outi_ref, fn_s, runv_s, runi_s):
    o = pl.program_id(0)
    i = pl.program_id(1)

    @pl.when(i == 0)
    def _init():
        f = feat_ref[...]
        nrm = jnp.sqrt(jnp.sum(f * f, axis=1, keepdims=True)) + 1e-12
        fn_s[...] = f / nrm
        runv_s[...] = jnp.full((B, 16), NEG, jnp.float32)
        runi_s[...] = jnp.zeros((B, 16), jnp.int32)

    w = w_ref[...]
    wn = w / (jnp.sqrt(jnp.sum(w * w, axis=1, keepdims=True)) + 1e-12)
    s = lax.dot_general(fn_s[...], wn, (((1,), (1,)), ((), ())),
                        preferred_element_type=jnp.float32)

    colio = lax.broadcasted_iota(jnp.int32, (B, BK), 1)
    lane16 = lax.broadcasted_iota(jnp.int32, (B, 16), 1)
    blk = o * IBLK + i
    for _ in range(NNBR):
        m = jnp.max(s, axis=1, keepdims=True)
        idx = jnp.min(jnp.where(s == m, colio, BK), axis=1, keepdims=True)
        s = jnp.where(colio == idx, NEG, s)
        gid = idx + blk * BK
        rv = runv_s[...]
        ri = runi_s[...]
        do = m > rv[:, 9:10]
        pos = jnp.sum((rv >= m).astype(jnp.int32), axis=1, keepdims=True)
        rv_shift = jnp.concatenate(
            [jnp.full((B, 1), NEG, jnp.float32), rv[:, :15]], axis=1)
        ri_shift = jnp.concatenate(
            [jnp.zeros((B, 1), jnp.int32), ri[:, :15]], axis=1)
        nrv = jnp.where(lane16 < pos, rv,
                        jnp.where(lane16 == pos, m, rv_shift))
        nri = jnp.where(lane16 < pos, ri,
                        jnp.where(lane16 == pos, gid, ri_shift))
        runv_s[...] = jnp.where(do, nrv, rv)
        runi_s[...] = jnp.where(do, nri, ri)

    @pl.when(i == IBLK - 1)
    def _emit():
        outv_ref[...] = runv_s[...].reshape(1, B, 16)
        outi_ref[...] = runi_s[...].reshape(1, B, 16)


def _run_topk(features, W):
    return pl.pallas_call(
        _topk_body,
        grid=(NCORE, IBLK),
        in_specs=[
            pl.BlockSpec((B, FDIM), lambda o, i: (0, 0)),
            pl.BlockSpec((BK, FDIM), lambda o, i: (o * IBLK + i, 0)),
        ],
        out_specs=[
            pl.BlockSpec((1, B, 16), lambda o, i: (o, 0, 0)),
            pl.BlockSpec((1, B, 16), lambda o, i: (o, 0, 0)),
        ],
        out_shape=[jax.ShapeDtypeStruct((NCORE, B, 16), jnp.float32),
                   jax.ShapeDtypeStruct((NCORE, B, 16), jnp.int32)],
        scratch_shapes=[
            pltpu.VMEM((B, FDIM), jnp.float32),
            pltpu.VMEM((B, 16), jnp.float32),
            pltpu.VMEM((B, 16), jnp.int32),
        ],
        compiler_params=pltpu.CompilerParams(
            dimension_semantics=("parallel", "arbitrary")),
    )(features, W)


def _merge_body(v_ref, i_ref, out_ref):
    catv = jnp.concatenate([v_ref[0], v_ref[1]], axis=1)
    cati = jnp.concatenate([i_ref[0], i_ref[1]], axis=1)
    cio = lax.broadcasted_iota(jnp.int32, (B, 32), 1)
    ni = []
    for _ in range(NNBR):
        m = jnp.max(catv, axis=1, keepdims=True)
        c = jnp.min(jnp.where(catv == m, cio, 32), axis=1, keepdims=True)
        hit = cio == c
        ni.append(jnp.sum(jnp.where(hit, cati, 0), axis=1, keepdims=True))
        catv = jnp.where(hit, NEG, catv)
    out_ref[...] = jnp.concatenate(
        ni + [jnp.zeros((B, 128 - NNBR), jnp.int32)], axis=1)


def _run_merge(vals, ids):
    return pl.pallas_call(
        _merge_body,
        out_shape=jax.ShapeDtypeStruct((B, 128), jnp.int32),
    )(vals, ids)


@functools.lru_cache(maxsize=None)
def _scatter_masks_kernel():
    return functools.partial(
        pl.kernel,
        mesh=_vmesh(),
        out_type=(jax.ShapeDtypeStruct((PAD_CLS,), jnp.float32),
                  jax.ShapeDtypeStruct((PAD_CLS,), jnp.float32)),
        scratch_types=[pltpu.VMEM((PAD_CLS,), jnp.float32),
                       pltpu.VMEM((B * NNBR,), jnp.int32)],
        compiler_params=_sc_params(),
    )(_scatter_masks_body)


def _scatter_masks_body(nbr_hbm, lab_hbm, ma_hbm, mb_hbm, mask_v, idx_v):
    cid = lax.axis_index("c")
    sid = lax.axis_index("s")
    zeros16 = jnp.zeros((16,), jnp.float32)
    ones16 = jnp.ones((16,), jnp.float32)

    @pl.when(jnp.logical_and(cid == 0, sid == 0))
    def _nbr_mask():
        @pl.loop(0, PAD_CLS, step=16)
        def _(j):
            mask_v[pl.ds(j, 16)] = zeros16

        pltpu.sync_copy(nbr_hbm, idx_v)

        @pl.loop(0, B * NNBR, step=16)
        def _(j):
            plsc.store_scatter(mask_v, [idx_v[pl.ds(j, 16)]], ones16)

        pltpu.sync_copy(mask_v, ma_hbm)

    @pl.when(jnp.logical_and(cid == 1, sid == 0))
    def _lab_mask():
        @pl.loop(0, PAD_CLS, step=16)
        def _(j):
            mask_v[pl.ds(j, 16)] = zeros16

        pltpu.sync_copy(lab_hbm, idx_v.at[pl.ds(0, B)])

        @pl.loop(0, B, step=16)
        def _(j):
            plsc.store_scatter(mask_v, [idx_v[pl.ds(j, 16)]], ones16)

        pltpu.sync_copy(mask_v, mb_hbm)


def _positions_body(ma_ref, mb_ref, pos_ref, val_ref, c2x_ref):
    m2 = mb_ref[...]
    m1 = ma_ref[...] * (1.0 - m2)
    r = lax.broadcasted_iota(jnp.int32, (128, 128), 0)
    c = lax.broadcasted_iota(jnp.int32, (128, 128), 1)
    upper = (r < c).astype(jnp.float32)
    rr = lax.broadcasted_iota(jnp.int32, (ROWS, ROWS), 0)
    cc = lax.broadcasted_iota(jnp.int32, (ROWS, ROWS), 1)
    lower = (cc < rr).astype(jnp.float32)

    def xcum(m):
        pre = lax.dot_general(m, upper, (((1,), (0,)), ((), ())),
                              preferred_element_type=jnp.float32)
        rs = jnp.sum(m, axis=1, keepdims=True)
        off = lax.dot_general(lower, rs, (((1,), (0,)), ((), ())),
                              preferred_element_type=jnp.float32)
        return pre + off

    c2 = xcum(m2)
    c1 = xcum(m1)
    n2 = jnp.sum(m2)
    n1 = jnp.sum(m1)
    ii = (lax.broadcasted_iota(jnp.int32, (ROWS, 128), 0) * 128
          + lax.broadcasted_iota(jnp.int32, (ROWS, 128), 1)).astype(jnp.float32)
    pos = jnp.where(m2 > 0.5, c2,
                    jnp.where(m1 > 0.5, n2 + c1, n2 + n1 + (ii - c2 - c1)))
    valid = jnp.logical_and(ii < float(NUM_CLS), pos < float(SAMP))
    pos_ref[...] = pos.astype(jnp.int32)
    val_ref[...] = valid.astype(jnp.int32)
    c2x_ref[...] = c2.astype(jnp.int32)


def _run_positions(maskA, maskB):
    return pl.pallas_call(
        _positions_body,
        out_shape=(jax.ShapeDtypeStruct((ROWS, 128), jnp.int32),
                   jax.ShapeDtypeStruct((ROWS, 128), jnp.int32),
                   jax.ShapeDtypeStruct((ROWS, 128), jnp.int32)),
    )(maskA.reshape(ROWS, 128), maskB.reshape(ROWS, 128))


@functools.lru_cache(maxsize=None)
def _compact_and_ranks_kernel():
    return functools.partial(
        pl.kernel,
        mesh=_vmesh(),
        out_type=(jax.ShapeDtypeStruct((SAMP,), jnp.int32),
                  jax.ShapeDtypeStruct((B,), jnp.int32)),
        scratch_types=[pltpu.VMEM((SAMP + 16,), jnp.int32),
                       pltpu.VMEM((CHUNK,), jnp.int32),
                       pltpu.VMEM((CHUNK,), jnp.int32),
                       pltpu.VMEM((PAD_CLS,), jnp.int32),
                       pltpu.VMEM((B,), jnp.int32),
                       pltpu.VMEM((B,), jnp.int32)],
        compiler_params=_sc_params(),
    )(_compact_and_ranks_body)


def _compact_and_ranks_body(pos_hbm, val_hbm, c2x_hbm, lab_hbm, sel_hbm,
                            idxs_hbm, sel_v, chp_v, chv_v, c2x_v, lab_v,
                            out_v):
    cid = lax.axis_index("c")
    sid = lax.axis_index("s")

    @pl.when(jnp.logical_and(cid == 0, sid == 0))
    def _compact():
        @pl.loop(0, SAMP + 16, step=16)
        def _(j):
            sel_v[pl.ds(j, 16)] = jnp.zeros((16,), jnp.int32)

        @pl.loop(0, 32)
        def _(ch):
            pltpu.sync_copy(pos_hbm.at[pl.ds(ch * CHUNK, CHUNK)], chp_v)
            pltpu.sync_copy(val_hbm.at[pl.ds(ch * CHUNK, CHUNK)], chv_v)

            @pl.loop(0, CHUNK, step=16)
            def _(k):
                p = jnp.minimum(chp_v[pl.ds(k, 16)], SAMP)
                ok = chv_v[pl.ds(k, 16)] > 0
                gid = (ch * CHUNK + k
                       + lax.broadcasted_iota(jnp.int32, (16,), 0))
                plsc.store_scatter(sel_v, [p], gid, mask=ok)

        pltpu.sync_copy(sel_v.at[pl.ds(0, SAMP)], sel_hbm)

    @pl.when(jnp.logical_and(cid == 1, sid == 0))
    def _ranks():
        pltpu.sync_copy(c2x_hbm, c2x_v)
        pltpu.sync_copy(lab_hbm, lab_v)

        @pl.loop(0, B, step=16)
        def _(k):
            out_v[pl.ds(k, 16)] = plsc.load_gather(
                c2x_v, [lab_v[pl.ds(k, 16)]])

        pltpu.sync_copy(out_v, idxs_hbm)


@functools.lru_cache(maxsize=None)
def _gather_rows_kernel():
    return functools.partial(
        pl.kernel,
        mesh=_vmesh(),
        out_type=jax.ShapeDtypeStruct((SAMP, FDIM), jnp.float32),
        scratch_types=[pltpu.VMEM((SAMP // 32,), jnp.int32),
                       pltpu.VMEM((SAMP // 32, FDIM), jnp.float32),
                       pltpu.SemaphoreType.DMA],
    )(_gather_rows_body)


def _gather_rows_body(sel_hbm, w_hbm, out_hbm, idx_v, rows_v, sem):
    wid = lax.axis_index("s") * 2 + lax.axis_index("c")
    base = wid * (SAMP // 32)
    pltpu.sync_copy(sel_hbm.at[pl.ds(base, SAMP // 32)], idx_v)
    pltpu.async_copy(w_hbm.at[idx_v], rows_v, sem).wait()
    pltpu.sync_copy(rows_v, out_hbm.at[pl.ds(base, SAMP // 32)])


def kernel(features, labels, W):
    Wp = jnp.pad(W, ((0, PAD_CLS - NUM_CLS), (0, 0)))
    scores, gm3 = _run_scores(features, Wp)
    gm = jnp.transpose(gm3, (1, 0, 2)).reshape(B, ROWS)
    flat, seg = _run_segtop(gm)
    cand = _gather_cand_kernel()(flat[:, :NNBR].reshape(-1),
                                 scores.reshape(-1, 128))
    ids_fast, flag = _run_cand(cand.reshape(B, NCAND), seg, gm)

    def _slow():
        vals, ids = _run_topk(features, W)
        return _run_merge(vals, ids)

    nbr_pad = lax.cond(flag[0, 0] == 0, lambda: ids_fast, _slow)
    nbrs = nbr_pad[:, :NNBR].reshape(-1)
    maskA, maskB = _scatter_masks_kernel()(nbrs, labels)
    pos, valid, c2x = _run_positions(maskA, maskB)
    sel, idxs = _compact_and_ranks_kernel()(pos.reshape(-1),
                                            valid.reshape(-1),
                                            c2x.reshape(-1), labels)
    weights = _gather_rows_kernel()(sel, W)
    bias = jnp.zeros((SAMP,), jnp.float32)
    return weights, bias, idxs


# speedup vs baseline: 4.0104x; 1.2138x over previous
import dataclasses
import functools

import jax
import jax.numpy as jnp
from jax import lax
from jax.experimental import pallas as pl
from jax.experimental.pallas import tpu as pltpu
from jax.experimental.pallas import tpu_sc as plsc

B = 1024
FDIM = 128
NUM_CLS = 100000
SAMP = 8192
NNBR = 10
PAD_CLS = 100352
ROWS = PAD_CLS // 128
NBLK = 50
BK = NUM_CLS // NBLK
CHUNK = PAD_CLS // 32
NEG = float(jnp.finfo(jnp.float32).min)

@functools.lru_cache(maxsize=None)
def _vmesh():
    return plsc.VectorSubcoreMesh(core_axis_name="c", subcore_axis_name="s")


@functools.lru_cache(maxsize=None)
def _sc_params():
    cp = pltpu.CompilerParams()
    if "needs_layout_passes" in pltpu.CompilerParams.__dataclass_fields__:
        cp = dataclasses.replace(cp, needs_layout_passes=False)
    return cp


BKA = 2048
NBLKA = PAD_CLS // BKA
SEGS_PER_BLK = BKA // 128
NCAND = NNBR * 128


def _scores_body(feat_ref, w_ref, sc_ref, gm_ref, fn_s):
    i = pl.program_id(0)

    @pl.when(i == 0)
    def _init():
        f = feat_ref[...]
        nrm = jnp.sqrt(jnp.sum(f * f, axis=1, keepdims=True)) + 1e-12
        fn_s[...] = f / nrm

    w = w_ref[...]
    wn = w / (jnp.sqrt(jnp.sum(w * w, axis=1, keepdims=True)) + 1e-12)
    s = lax.dot_general(fn_s[...], wn, (((1,), (1,)), ((), ())),
                        preferred_element_type=jnp.float32)
    gcol = lax.broadcasted_iota(jnp.int32, (B, BKA), 1) + i * BKA
    s = jnp.where(gcol < NUM_CLS, s, NEG)
    sc_ref[...] = s
    gm_ref[...] = jnp.max(s.reshape(B, SEGS_PER_BLK, 128),
                          axis=2).reshape(1, B, SEGS_PER_BLK)


def _run_scores(features, Wp):
    return pl.pallas_call(
        _scores_body,
        grid=(NBLKA,),
        in_specs=[
            pl.BlockSpec((B, FDIM), lambda i: (0, 0)),
            pl.BlockSpec((BKA, FDIM), lambda i: (i, 0)),
        ],
        out_specs=[
            pl.BlockSpec((B, BKA), lambda i: (0, i)),
            pl.BlockSpec((1, B, SEGS_PER_BLK), lambda i: (i, 0, 0)),
        ],
        out_shape=[jax.ShapeDtypeStruct((B, PAD_CLS), jnp.float32),
                   jax.ShapeDtypeStruct((NBLKA, B, SEGS_PER_BLK),
                                        jnp.float32)],
        scratch_shapes=[pltpu.VMEM((B, FDIM), jnp.float32)],
    )(features, Wp)


def _segtop_body(gm_ref, flat_ref, seg_ref):
    gm = gm_ref[...]
    colio = lax.broadcasted_iota(jnp.int32, (B, ROWS), 1)
    segs = []
    for _ in range(NNBR):
        m = jnp.max(gm, axis=1, keepdims=True)
        sid = jnp.min(jnp.where(gm == m, colio, ROWS), axis=1, keepdims=True)
        gm = jnp.where(colio == sid, NEG, gm)
        segs.append(sid)
    seg = jnp.concatenate(segs + [jnp.zeros((B, 6), jnp.int32)], axis=1)
    seg_ref[...] = seg
    rowio = lax.broadcasted_iota(jnp.int32, (B, 16), 0)
    lane16 = lax.broadcasted_iota(jnp.int32, (B, 16), 1)
    flat_ref[...] = jnp.where(lane16 < NNBR, rowio * ROWS + seg, 0)


def _run_segtop(gm):
    return pl.pallas_call(
        _segtop_body,
        out_shape=[jax.ShapeDtypeStruct((B, 16), jnp.int32),
                   jax.ShapeDtypeStruct((B, 16), jnp.int32)],
    )(gm)


@functools.lru_cache(maxsize=None)
def _gather_cand_kernel():
    n_per = B * NNBR // 32

    @functools.partial(
        pl.kernel,
        mesh=_vmesh(),
        out_type=jax.ShapeDtypeStruct((B * NNBR, 128), jnp.float32),
        scratch_types=[pltpu.VMEM((n_per,), jnp.int32),
                       pltpu.VMEM((n_per, 128), jnp.float32),
                       pltpu.SemaphoreType.DMA],
    )
    def gather_cand(flat_hbm, table_hbm, out_hbm, idx_v, rows_v, sem):
        wid = lax.axis_index("s") * 2 + lax.axis_index("c")
        base = wid * n_per
        pltpu.sync_copy(flat_hbm.at[pl.ds(base, n_per)], idx_v)
        pltpu.async_copy(table_hbm.at[idx_v], rows_v, sem).wait()
        pltpu.sync_copy(rows_v, out_hbm.at[pl.ds(base, n_per)])

    return gather_cand


def _cand_body(cand_ref, seg_ref, gm_ref, ids_ref, flag_ref):
    s = cand_ref[...]
    colio = lax.broadcasted_iota(jnp.int32, (B, NCAND), 1)
    seg = seg_ref[...]
    l16 = lax.broadcasted_iota(jnp.int32, (B, 16), 1)
    ids = []
    lastm = None
    for _ in range(NNBR):
        m = jnp.max(s, axis=1, keepdims=True)
        c = jnp.min(jnp.where(s == m, colio, NCAND), axis=1, keepdims=True)
        s = jnp.where(colio == c, NEG, s)
        r = c // 128
        w = c - r * 128
        segsel = jnp.sum(jnp.where(l16 == r, seg, 0), axis=1, keepdims=True)
        ids.append(segsel * 128 + w)
        lastm = m
    ids_ref[...] = jnp.concatenate(
        ids + [jnp.zeros((B, 128 - NNBR), jnp.int32)], axis=1)
    gm = gm_ref[...]
    cnt = jnp.sum((gm >= lastm).astype(jnp.int32), axis=1)
    flag_ref[...] = jnp.reshape(jnp.sum((cnt > NNBR).astype(jnp.int32)),
                                (1, 1))


def _run_cand(cand, seg, gm):
    return pl.pallas_call(
        _cand_body,
        out_shape=[jax.ShapeDtypeStruct((B, 128), jnp.int32),
                   jax.ShapeDtypeStruct((1, 1), jnp.int32)],
    )(cand, seg, gm)


NCORE = 2
IBLK = NBLK // NCORE


def _topk_body(feat_ref, w_ref, outv_ref, outi_ref, fn_s, runv_s, runi_s):
    o = pl.program_id(0)
    i = pl.program_id(1)

    @pl.when(i == 0)
    def _init():
        f = feat_ref[...]
        nrm = jnp.sqrt(jnp.sum(f * f, axis=1, keepdims=True)) + 1e-12
        fn_s[...] = f / nrm
        runv_s[...] = jnp.full((B, 16), NEG, jnp.float32)
        runi_s[...] = jnp.zeros((B, 16), jnp.int32)

    w = w_ref[...]
    wn = w / (jnp.sqrt(jnp.sum(w * w, axis=1, keepdims=True)) + 1e-12)
    s = lax.dot_general(fn_s[...], wn, (((1,), (1,)), ((), ())),
                        preferred_element_type=jnp.float32)

    colio = lax.broadcasted_iota(jnp.int32, (B, BK), 1)
    lane16 = lax.broadcasted_iota(jnp.int32, (B, 16), 1)
    blk = o * IBLK + i
    for _ in range(NNBR):
        m = jnp.max(s, axis=1, keepdims=True)
        idx = jnp.min(jnp.where(s == m, colio, BK), axis=1, keepdims=True)
        s = jnp.where(colio == idx, NEG, s)
        gid = idx + blk * BK
        rv = runv_s[...]
        ri = runi_s[...]
        do = m > rv[:, 9:10]
        pos = jnp.sum((rv >= m).astype(jnp.int32), axis=1, keepdims=True)
        rv_shift = jnp.concatenate(
            [jnp.full((B, 1), NEG, jnp.float32), rv[:, :15]], axis=1)
        ri_shift = jnp.concatenate(
            [jnp.zeros((B, 1), jnp.int32), ri[:, :15]], axis=1)
        nrv = jnp.where(lane16 < pos, rv,
                        jnp.where(lane16 == pos, m, rv_shift))
        nri = jnp.where(lane16 < pos, ri,
                        jnp.where(lane16 == pos, gid, ri_shift))
        runv_s[...] = jnp.where(do, nrv, rv)
        runi_s[...] = jnp.where(do, nri, ri)

    @pl.when(i == IBLK - 1)
    def _emit():
        outv_ref[...] = runv_s[...].reshape(1, B, 16)
        outi_ref[...] = runi_s[...].reshape(1, B, 16)


def _run_topk(features, W):
    return pl.pallas_call(
        _topk_body,
        grid=(NCORE, IBLK),
        in_specs=[
            pl.BlockSpec((B, FDIM), lambda o, i: (0, 0)),
            pl.BlockSpec((BK, FDIM), lambda o, i: (o * IBLK + i, 0)),
        ],
        out_specs=[
            pl.BlockSpec((1, B, 16), lambda o, i: (o, 0, 0)),
            pl.BlockSpec((1, B, 16), lambda o, i: (o, 0, 0)),
        ],
        out_shape=[jax.ShapeDtypeStruct((NCORE, B, 16), jnp.float32),
                   jax.ShapeDtypeStruct((NCORE, B, 16), jnp.int32)],
        scratch_shapes=[
            pltpu.VMEM((B, FDIM), jnp.float32),
            pltpu.VMEM((B, 16), jnp.float32),
            pltpu.VMEM((B, 16), jnp.int32),
        ],
        compiler_params=pltpu.CompilerParams(
            dimension_semantics=("parallel", "arbitrary")),
    )(features, W)


def _merge_body(v_ref, i_ref, out_ref):
    catv = jnp.concatenate([v_ref[0], v_ref[1]], axis=1)
    cati = jnp.concatenate([i_ref[0], i_ref[1]], axis=1)
    cio = lax.broadcasted_iota(jnp.int32, (B, 32), 1)
    ni = []
    for _ in range(NNBR):
        m = jnp.max(catv, axis=1, keepdims=True)
        c = jnp.min(jnp.where(catv == m, cio, 32), axis=1, keepdims=True)
        hit = cio == c
        ni.append(jnp.sum(jnp.where(hit, cati, 0), axis=1, keepdims=True))
        catv = jnp.where(hit, NEG, catv)
    out_ref[...] = jnp.concatenate(
        ni + [jnp.zeros((B, 128 - NNBR), jnp.int32)], axis=1)


def _run_merge(vals, ids):
    return pl.pallas_call(
        _merge_body,
        out_shape=jax.ShapeDtypeStruct((B, 128), jnp.int32),
    )(vals, ids)


@functools.lru_cache(maxsize=None)
def _scatter_masks_kernel():
    return functools.partial(
        pl.kernel,
        mesh=_vmesh(),
        out_type=(jax.ShapeDtypeStruct((PAD_CLS,), jnp.float32),
                  jax.ShapeDtypeStruct((PAD_CLS,), jnp.float32)),
        scratch_types=[pltpu.VMEM((PAD_CLS,), jnp.float32),
                       pltpu.VMEM((B * NNBR,), jnp.int32)],
        compiler_params=_sc_params(),
    )(_scatter_masks_body)


def _scatter_masks_body(zer_hbm, nbr_hbm, lab_hbm, ma_hbm, mb_hbm, mask_v,
                        idx_v):
    cid = lax.axis_index("c")
    sid = lax.axis_index("s")
    ones16 = jnp.ones((16,), jnp.float32)

    @pl.when(jnp.logical_and(cid == 0, sid == 0))
    def _nbr_mask():
        pltpu.sync_copy(zer_hbm, mask_v)
        pltpu.sync_copy(nbr_hbm, idx_v)

        @pl.loop(0, B * NNBR, step=16)
        def _(j):
            plsc.store_scatter(mask_v, [idx_v[pl.ds(j, 16)]], ones16)

        pltpu.sync_copy(mask_v, ma_hbm)

    @pl.when(jnp.logical_and(cid == 1, sid == 0))
    def _lab_mask():
        pltpu.sync_copy(zer_hbm, mask_v)
        pltpu.sync_copy(lab_hbm, idx_v.at[pl.ds(0, B)])

        @pl.loop(0, B, step=16)
        def _(j):
            plsc.store_scatter(mask_v, [idx_v[pl.ds(j, 16)]], ones16)

        pltpu.sync_copy(mask_v, mb_hbm)


def _positions_body(ma_ref, mb_ref, pos_ref, val_ref, c2x_ref):
    m2 = mb_ref[...]
    m1 = ma_ref[...] * (1.0 - m2)
    r = lax.broadcasted_iota(jnp.int32, (128, 128), 0)
    c = lax.broadcasted_iota(jnp.int32, (128, 128), 1)
    upper = (r < c).astype(jnp.float32)
    rr = lax.broadcasted_iota(jnp.int32, (ROWS, ROWS), 0)
    cc = lax.broadcasted_iota(jnp.int32, (ROWS, ROWS), 1)
    lower = (cc < rr).astype(jnp.float32)

    def xcum(m):
        pre = lax.dot_general(m, upper, (((1,), (0,)), ((), ())),
                              preferred_element_type=jnp.float32)
        rs = jnp.sum(m, axis=1, keepdims=True)
        off = lax.dot_general(lower, rs, (((1,), (0,)), ((), ())),
                              preferred_element_type=jnp.float32)
        return pre + off

    c2 = xcum(m2)
    c1 = xcum(m1)
    n2 = jnp.sum(m2)
    n1 = jnp.sum(m1)
    ii = (lax.broadcasted_iota(jnp.int32, (ROWS, 128), 0) * 128
          + lax.broadcasted_iota(jnp.int32, (ROWS, 128), 1)).astype(jnp.float32)
    pos = jnp.where(m2 > 0.5, c2,
                    jnp.where(m1 > 0.5, n2 + c1, n2 + n1 + (ii - c2 - c1)))
    valid = jnp.logical_and(ii < float(NUM_CLS), pos < float(SAMP))
    pos_ref[...] = pos.astype(jnp.int32)
    val_ref[...] = valid.astype(jnp.int32)
    c2x_ref[...] = c2.astype(jnp.int32)


def _run_positions(maskA, maskB):
    return pl.pallas_call(
        _positions_body,
        out_shape=(jax.ShapeDtypeStruct((ROWS, 128), jnp.int32),
                   jax.ShapeDtypeStruct((ROWS, 128), jnp.int32),
                   jax.ShapeDtypeStruct((ROWS, 128), jnp.int32)),
    )(maskA.reshape(ROWS, 128), maskB.reshape(ROWS, 128))


SAMP_PAD = SAMP + 128


@functools.lru_cache(maxsize=None)
def _compact_and_ranks_kernel():
    return functools.partial(
        pl.kernel,
        mesh=_vmesh(),
        out_type=(jax.ShapeDtypeStruct((32, SAMP_PAD), jnp.int32),
                  jax.ShapeDtypeStruct((B,), jnp.int32)),
        scratch_types=[pltpu.VMEM((SAMP_PAD,), jnp.int32),
                       pltpu.VMEM((CHUNK,), jnp.int32),
                       pltpu.VMEM((CHUNK,), jnp.int32),
                       pltpu.VMEM((PAD_CLS,), jnp.int32),
                       pltpu.VMEM((B,), jnp.int32),
                       pltpu.VMEM((B,), jnp.int32)],
        compiler_params=_sc_params(),
    )(_compact_and_ranks_body)


def _compact_and_ranks_body(pos_hbm, val_hbm, c2x_hbm, lab_hbm, selp_hbm,
                            idxs_hbm, sel_v, chp_v, chv_v, c2x_v, lab_v,
                            out_v):
    cid = lax.axis_index("c")
    sid = lax.axis_index("s")
    wid = sid * 2 + cid
    base = wid * CHUNK

    @pl.loop(0, SAMP_PAD, step=16)
    def _(j):
        sel_v[pl.ds(j, 16)] = jnp.zeros((16,), jnp.int32)

    pltpu.sync_copy(pos_hbm.at[pl.ds(base, CHUNK)], chp_v)
    pltpu.sync_copy(val_hbm.at[pl.ds(base, CHUNK)], chv_v)

    @pl.loop(0, CHUNK, step=16)
    def _(k):
        p = jnp.minimum(chp_v[pl.ds(k, 16)], SAMP)
        ok = chv_v[pl.ds(k, 16)] > 0
        gid = base + k + 1 + lax.broadcasted_iota(jnp.int32, (16,), 0)
        plsc.store_scatter(sel_v, [p], gid, mask=ok)

    pltpu.sync_copy(sel_v, selp_hbm.at[wid])

    @pl.when(jnp.logical_and(cid == 1, sid == 0))
    def _ranks():
        pltpu.sync_copy(c2x_hbm, c2x_v)
        pltpu.sync_copy(lab_hbm, lab_v)

        @pl.loop(0, B, step=16)
        def _(k):
            out_v[pl.ds(k, 16)] = plsc.load_gather(
                c2x_v, [lab_v[pl.ds(k, 16)]])

        pltpu.sync_copy(out_v, idxs_hbm)


def _selmerge_body(in_ref, out_ref):
    x = in_ref[...]
    out_ref[...] = jnp.sum(x[:, :SAMP // 128, :], axis=0) - 1


def _run_selmerge(selp):
    return pl.pallas_call(
        _selmerge_body,
        out_shape=jax.ShapeDtypeStruct((SAMP // 128, 128), jnp.int32),
    )(selp.reshape(32, SAMP_PAD // 128, 128))


@functools.lru_cache(maxsize=None)
def _gather_rows_kernel():
    return functools.partial(
        pl.kernel,
        mesh=_vmesh(),
        out_type=jax.ShapeDtypeStruct((SAMP, FDIM), jnp.float32),
        scratch_types=[pltpu.VMEM((SAMP // 32,), jnp.int32),
                       pltpu.VMEM((SAMP // 32, FDIM), jnp.float32),
                       pltpu.SemaphoreType.DMA],
    )(_gather_rows_body)


def _gather_rows_body(sel_hbm, w_hbm, out_hbm, idx_v, rows_v, sem):
    wid = lax.axis_index("s") * 2 + lax.axis_index("c")
    base = wid * (SAMP // 32)
    pltpu.sync_copy(sel_hbm.at[pl.ds(base, SAMP // 32)], idx_v)
    pltpu.async_copy(w_hbm.at[idx_v], rows_v, sem).wait()
    pltpu.sync_copy(rows_v, out_hbm.at[pl.ds(base, SAMP // 32)])


def kernel(features, labels, W):
    scores, gm3 = _run_scores(features, W)
    gm = jnp.transpose(gm3, (1, 0, 2)).reshape(B, ROWS)
    flat, seg = _run_segtop(gm)
    cand = _gather_cand_kernel()(flat[:, :NNBR].reshape(-1),
                                 scores.reshape(-1, 128))
    ids_fast, flag = _run_cand(cand.reshape(B, NCAND), seg, gm)

    def _slow():
        vals, ids = _run_topk(features, W)
        return _run_merge(vals, ids)

    nbr_pad = lax.cond(flag[0, 0] == 0, lambda: ids_fast, _slow)
    nbrs = nbr_pad[:, :NNBR].reshape(-1)
    zer = jnp.zeros((PAD_CLS,), jnp.float32)
    maskA, maskB = _scatter_masks_kernel()(zer, nbrs, labels)
    pos, valid, c2x = _run_positions(maskA, maskB)
    selp, idxs = _compact_and_ranks_kernel()(pos.reshape(-1),
                                             valid.reshape(-1),
                                             c2x.reshape(-1), labels)
    sel = _run_selmerge(selp).reshape(-1)
    weights = _gather_rows_kernel()(sel, W)
    bias = jnp.zeros((SAMP,), jnp.float32)
    return weights, bias, idxs


# speedup vs baseline: 4.0878x; 1.0193x over previous
import dataclasses
import functools

import jax
import jax.numpy as jnp
from jax import lax
from jax.experimental import pallas as pl
from jax.experimental.pallas import tpu as pltpu
from jax.experimental.pallas import tpu_sc as plsc

B = 1024
FDIM = 128
NUM_CLS = 100000
SAMP = 8192
NNBR = 10
PAD_CLS = 100352
ROWS = PAD_CLS // 128
NBLK = 50
BK = NUM_CLS // NBLK
CHUNK = PAD_CLS // 32
NEG = float(jnp.finfo(jnp.float32).min)

@functools.lru_cache(maxsize=None)
def _vmesh():
    return plsc.VectorSubcoreMesh(core_axis_name="c", subcore_axis_name="s")


@functools.lru_cache(maxsize=None)
def _sc_params():
    cp = pltpu.CompilerParams()
    if "needs_layout_passes" in pltpu.CompilerParams.__dataclass_fields__:
        cp = dataclasses.replace(cp, needs_layout_passes=False)
    return cp


BKA = 2048
NBLKA = PAD_CLS // BKA
SEGS_PER_BLK = BKA // 128
NCAND = NNBR * 128


def _scores_body(feat_ref, w_ref, sc_ref, gm_ref, fn_s):
    i = pl.program_id(0)

    @pl.when(i == 0)
    def _init():
        f = feat_ref[...]
        nrm = jnp.sqrt(jnp.sum(f * f, axis=1, keepdims=True)) + 1e-12
        fn_s[...] = f / nrm

    w = w_ref[...]
    wn = w / (jnp.sqrt(jnp.sum(w * w, axis=1, keepdims=True)) + 1e-12)
    s = lax.dot_general(fn_s[...], wn, (((1,), (1,)), ((), ())),
                        preferred_element_type=jnp.float32)

    @pl.when(i < NBLKA - 1)
    def _plain():
        sc_ref[...] = s
        gm_ref[...] = jnp.max(s.reshape(B, SEGS_PER_BLK, 128),
                              axis=2).reshape(1, B, SEGS_PER_BLK)

    @pl.when(i == NBLKA - 1)
    def _masked():
        gcol = lax.broadcasted_iota(jnp.int32, (B, BKA), 1) + i * BKA
        sm = jnp.where(gcol < NUM_CLS, s, NEG)
        sc_ref[...] = sm
        gm_ref[...] = jnp.max(sm.reshape(B, SEGS_PER_BLK, 128),
                              axis=2).reshape(1, B, SEGS_PER_BLK)


def _run_scores(features, Wp):
    return pl.pallas_call(
        _scores_body,
        grid=(NBLKA,),
        in_specs=[
            pl.BlockSpec((B, FDIM), lambda i: (0, 0)),
            pl.BlockSpec((BKA, FDIM), lambda i: (i, 0)),
        ],
        out_specs=[
            pl.BlockSpec((B, BKA), lambda i: (0, i)),
            pl.BlockSpec((1, B, SEGS_PER_BLK), lambda i: (i, 0, 0)),
        ],
        out_shape=[jax.ShapeDtypeStruct((B, PAD_CLS), jnp.float32),
                   jax.ShapeDtypeStruct((NBLKA, B, SEGS_PER_BLK),
                                        jnp.float32)],
        scratch_shapes=[pltpu.VMEM((B, FDIM), jnp.float32)],
    )(features, Wp)


def _segtop_body(gm_ref, flat_ref, seg_ref):
    gm = gm_ref[...]
    colio = lax.broadcasted_iota(jnp.int32, (B, ROWS), 1)
    segs = []
    for _ in range(NNBR):
        m = jnp.max(gm, axis=1, keepdims=True)
        sid = jnp.min(jnp.where(gm == m, colio, ROWS), axis=1, keepdims=True)
        gm = jnp.where(colio == sid, NEG, gm)
        segs.append(sid)
    seg = jnp.concatenate(segs + [jnp.zeros((B, 6), jnp.int32)], axis=1)
    seg_ref[...] = seg
    rowio = lax.broadcasted_iota(jnp.int32, (B, 16), 0)
    lane16 = lax.broadcasted_iota(jnp.int32, (B, 16), 1)
    flat_ref[...] = jnp.where(lane16 < NNBR, rowio * ROWS + seg, 0)


def _run_segtop(gm):
    return pl.pallas_call(
        _segtop_body,
        out_shape=[jax.ShapeDtypeStruct((B, 16), jnp.int32),
                   jax.ShapeDtypeStruct((B, 16), jnp.int32)],
    )(gm)


@functools.lru_cache(maxsize=None)
def _gather_cand_kernel():
    n_per = B * NNBR // 32

    @functools.partial(
        pl.kernel,
        mesh=_vmesh(),
        out_type=jax.ShapeDtypeStruct((B * NNBR, 128), jnp.float32),
        scratch_types=[pltpu.VMEM((n_per,), jnp.int32),
                       pltpu.VMEM((n_per, 128), jnp.float32),
                       pltpu.SemaphoreType.DMA],
    )
    def gather_cand(flat_hbm, table_hbm, out_hbm, idx_v, rows_v, sem):
        wid = lax.axis_index("s") * 2 + lax.axis_index("c")
        base = wid * n_per
        pltpu.sync_copy(flat_hbm.at[pl.ds(base, n_per)], idx_v)
        pltpu.async_copy(table_hbm.at[idx_v], rows_v, sem).wait()
        pltpu.sync_copy(rows_v, out_hbm.at[pl.ds(base, n_per)])

    return gather_cand


def _cand_body(cand_ref, seg_ref, gm_ref, ids_ref, flag_ref):
    s = cand_ref[...]
    colio = lax.broadcasted_iota(jnp.int32, (B, NCAND), 1)
    seg = seg_ref[...]
    l16 = lax.broadcasted_iota(jnp.int32, (B, 16), 1)
    ids = []
    lastm = None
    for _ in range(NNBR):
        m = jnp.max(s, axis=1, keepdims=True)
        c = jnp.min(jnp.where(s == m, colio, NCAND), axis=1, keepdims=True)
        s = jnp.where(colio == c, NEG, s)
        r = c // 128
        w = c - r * 128
        segsel = jnp.sum(jnp.where(l16 == r, seg, 0), axis=1, keepdims=True)
        ids.append(segsel * 128 + w)
        lastm = m
    ids_ref[...] = jnp.concatenate(
        ids + [jnp.zeros((B, 128 - NNBR), jnp.int32)], axis=1)
    gm = gm_ref[...]
    cnt = jnp.sum((gm >= lastm).astype(jnp.int32), axis=1)
    flag_ref[...] = jnp.reshape(jnp.sum((cnt > NNBR).astype(jnp.int32)),
                                (1, 1))


def _run_cand(cand, seg, gm):
    return pl.pallas_call(
        _cand_body,
        out_shape=[jax.ShapeDtypeStruct((B, 128), jnp.int32),
                   jax.ShapeDtypeStruct((1, 1), jnp.int32)],
    )(cand, seg, gm)


NCORE = 2
IBLK = NBLK // NCORE


def _topk_body(feat_ref, w_ref, outv_ref, outi_ref, fn_s, runv_s, runi_s):
    o = pl.program_id(0)
    i = pl.program_id(1)

    @pl.when(i == 0)
    def _init():
        f = feat_ref[...]
        nrm = jnp.sqrt(jnp.sum(f * f, axis=1, keepdims=True)) + 1e-12
        fn_s[...] = f / nrm
        runv_s[...] = jnp.full((B, 16), NEG, jnp.float32)
        runi_s[...] = jnp.zeros((B, 16), jnp.int32)

    w = w_ref[...]
    wn = w / (jnp.sqrt(jnp.sum(w * w, axis=1, keepdims=True)) + 1e-12)
    s = lax.dot_general(fn_s[...], wn, (((1,), (1,)), ((), ())),
                        preferred_element_type=jnp.float32)

    colio = lax.broadcasted_iota(jnp.int32, (B, BK), 1)
    lane16 = lax.broadcasted_iota(jnp.int32, (B, 16), 1)
    blk = o * IBLK + i
    for _ in range(NNBR):
        m = jnp.max(s, axis=1, keepdims=True)
        idx = jnp.min(jnp.where(s == m, colio, BK), axis=1, keepdims=True)
        s = jnp.where(colio == idx, NEG, s)
        gid = idx + blk * BK
        rv = runv_s[...]
        ri = runi_s[...]
        do = m > rv[:, 9:10]
        pos = jnp.sum((rv >= m).astype(jnp.int32), axis=1, keepdims=True)
        rv_shift = jnp.concatenate(
            [jnp.full((B, 1), NEG, jnp.float32), rv[:, :15]], axis=1)
        ri_shift = jnp.concatenate(
            [jnp.zeros((B, 1), jnp.int32), ri[:, :15]], axis=1)
        nrv = jnp.where(lane16 < pos, rv,
                        jnp.where(lane16 == pos, m, rv_shift))
        nri = jnp.where(lane16 < pos, ri,
                        jnp.where(lane16 == pos, gid, ri_shift))
        runv_s[...] = jnp.where(do, nrv, rv)
        runi_s[...] = jnp.where(do, nri, ri)

    @pl.when(i == IBLK - 1)
    def _emit():
        outv_ref[...] = runv_s[...].reshape(1, B, 16)
        outi_ref[...] = runi_s[...].reshape(1, B, 16)


def _run_topk(features, W):
    return pl.pallas_call(
        _topk_body,
        grid=(NCORE, IBLK),
        in_specs=[
            pl.BlockSpec((B, FDIM), lambda o, i: (0, 0)),
            pl.BlockSpec((BK, FDIM), lambda o, i: (o * IBLK + i, 0)),
        ],
        out_specs=[
            pl.BlockSpec((1, B, 16), lambda o, i: (o, 0, 0)),
            pl.BlockSpec((1, B, 16), lambda o, i: (o, 0, 0)),
        ],
        out_shape=[jax.ShapeDtypeStruct((NCORE, B, 16), jnp.float32),
                   jax.ShapeDtypeStruct((NCORE, B, 16), jnp.int32)],
        scratch_shapes=[
            pltpu.VMEM((B, FDIM), jnp.float32),
            pltpu.VMEM((B, 16), jnp.float32),
            pltpu.VMEM((B, 16), jnp.int32),
        ],
        compiler_params=pltpu.CompilerParams(
            dimension_semantics=("parallel", "arbitrary")),
    )(features, W)


def _merge_body(v_ref, i_ref, out_ref):
    catv = jnp.concatenate([v_ref[0], v_ref[1]], axis=1)
    cati = jnp.concatenate([i_ref[0], i_ref[1]], axis=1)
    cio = lax.broadcasted_iota(jnp.int32, (B, 32), 1)
    ni = []
    for _ in range(NNBR):
        m = jnp.max(catv, axis=1, keepdims=True)
        c = jnp.min(jnp.where(catv == m, cio, 32), axis=1, keepdims=True)
        hit = cio == c
        ni.append(jnp.sum(jnp.where(hit, cati, 0), axis=1, keepdims=True))
        catv = jnp.where(hit, NEG, catv)
    out_ref[...] = jnp.concatenate(
        ni + [jnp.zeros((B, 128 - NNBR), jnp.int32)], axis=1)


def _run_merge(vals, ids):
    return pl.pallas_call(
        _merge_body,
        out_shape=jax.ShapeDtypeStruct((B, 128), jnp.int32),
    )(vals, ids)


@functools.lru_cache(maxsize=None)
def _scatter_masks_kernel():
    return functools.partial(
        pl.kernel,
        mesh=_vmesh(),
        out_type=(jax.ShapeDtypeStruct((PAD_CLS,), jnp.float32),
                  jax.ShapeDtypeStruct((PAD_CLS,), jnp.float32)),
        scratch_types=[pltpu.VMEM((PAD_CLS,), jnp.float32),
                       pltpu.VMEM((B * NNBR,), jnp.int32)],
        compiler_params=_sc_params(),
    )(_scatter_masks_body)


def _scatter_masks_body(zer_hbm, nbr_hbm, lab_hbm, ma_hbm, mb_hbm, mask_v,
                        idx_v):
    cid = lax.axis_index("c")
    sid = lax.axis_index("s")
    ones16 = jnp.ones((16,), jnp.float32)

    @pl.when(jnp.logical_and(cid == 0, sid == 0))
    def _nbr_mask():
        pltpu.sync_copy(zer_hbm, mask_v)
        pltpu.sync_copy(nbr_hbm, idx_v)

        @pl.loop(0, B * NNBR, step=16)
        def _(j):
            plsc.store_scatter(mask_v, [idx_v[pl.ds(j, 16)]], ones16)

        pltpu.sync_copy(mask_v, ma_hbm)

    @pl.when(jnp.logical_and(cid == 1, sid == 0))
    def _lab_mask():
        pltpu.sync_copy(zer_hbm, mask_v)
        pltpu.sync_copy(lab_hbm, idx_v.at[pl.ds(0, B)])

        @pl.loop(0, B, step=16)
        def _(j):
            plsc.store_scatter(mask_v, [idx_v[pl.ds(j, 16)]], ones16)

        pltpu.sync_copy(mask_v, mb_hbm)


def _positions_body(ma_ref, mb_ref, pos_ref, val_ref, c2x_ref):
    m2 = mb_ref[...]
    m1 = ma_ref[...] * (1.0 - m2)
    r = lax.broadcasted_iota(jnp.int32, (128, 128), 0)
    c = lax.broadcasted_iota(jnp.int32, (128, 128), 1)
    upper = (r < c).astype(jnp.float32)
    rr = lax.broadcasted_iota(jnp.int32, (ROWS, ROWS), 0)
    cc = lax.broadcasted_iota(jnp.int32, (ROWS, ROWS), 1)
    lower = (cc < rr).astype(jnp.float32)

    def xcum(m):
        pre = lax.dot_general(m, upper, (((1,), (0,)), ((), ())),
                              preferred_element_type=jnp.float32)
        rs = jnp.sum(m, axis=1, keepdims=True)
        off = lax.dot_general(lower, rs, (((1,), (0,)), ((), ())),
                              preferred_element_type=jnp.float32)
        return pre + off

    c2 = xcum(m2)
    c1 = xcum(m1)
    n2 = jnp.sum(m2)
    n1 = jnp.sum(m1)
    ii = (lax.broadcasted_iota(jnp.int32, (ROWS, 128), 0) * 128
          + lax.broadcasted_iota(jnp.int32, (ROWS, 128), 1)).astype(jnp.float32)
    pos = jnp.where(m2 > 0.5, c2,
                    jnp.where(m1 > 0.5, n2 + c1, n2 + n1 + (ii - c2 - c1)))
    valid = jnp.logical_and(ii < float(NUM_CLS), pos < float(SAMP))
    pos_ref[...] = pos.astype(jnp.int32)
    val_ref[...] = valid.astype(jnp.int32)
    c2x_ref[...] = c2.astype(jnp.int32)


def _run_positions(maskA, maskB):
    return pl.pallas_call(
        _positions_body,
        out_shape=(jax.ShapeDtypeStruct((ROWS, 128), jnp.int32),
                   jax.ShapeDtypeStruct((ROWS, 128), jnp.int32),
                   jax.ShapeDtypeStruct((ROWS, 128), jnp.int32)),
    )(maskA.reshape(ROWS, 128), maskB.reshape(ROWS, 128))


SAMP_PAD = SAMP + 128


@functools.lru_cache(maxsize=None)
def _compact_and_ranks_kernel():
    return functools.partial(
        pl.kernel,
        mesh=_vmesh(),
        out_type=(jax.ShapeDtypeStruct((32, SAMP_PAD), jnp.int32),
                  jax.ShapeDtypeStruct((B,), jnp.int32)),
        scratch_types=[pltpu.VMEM((SAMP_PAD,), jnp.int32),
                       pltpu.VMEM((CHUNK,), jnp.int32),
                       pltpu.VMEM((CHUNK,), jnp.int32),
                       pltpu.VMEM((PAD_CLS,), jnp.int32),
                       pltpu.VMEM((B,), jnp.int32),
                       pltpu.VMEM((B,), jnp.int32)],
        compiler_params=_sc_params(),
    )(_compact_and_ranks_body)


def _compact_and_ranks_body(pos_hbm, val_hbm, c2x_hbm, lab_hbm, selp_hbm,
                            idxs_hbm, sel_v, chp_v, chv_v, c2x_v, lab_v,
                            out_v):
    cid = lax.axis_index("c")
    sid = lax.axis_index("s")
    wid = sid * 2 + cid
    base = wid * CHUNK

    @pl.loop(0, SAMP_PAD, step=16)
    def _(j):
        sel_v[pl.ds(j, 16)] = jnp.zeros((16,), jnp.int32)

    pltpu.sync_copy(pos_hbm.at[pl.ds(base, CHUNK)], chp_v)
    pltpu.sync_copy(val_hbm.at[pl.ds(base, CHUNK)], chv_v)

    @pl.loop(0, CHUNK, step=16)
    def _(k):
        p = jnp.minimum(chp_v[pl.ds(k, 16)], SAMP)
        ok = chv_v[pl.ds(k, 16)] > 0
        gid = base + k + 1 + lax.broadcasted_iota(jnp.int32, (16,), 0)
        plsc.store_scatter(sel_v, [p], gid, mask=ok)

    pltpu.sync_copy(sel_v, selp_hbm.at[wid])

    @pl.when(jnp.logical_and(cid == 1, sid == 0))
    def _ranks():
        pltpu.sync_copy(c2x_hbm, c2x_v)
        pltpu.sync_copy(lab_hbm, lab_v)

        @pl.loop(0, B, step=16)
        def _(k):
            out_v[pl.ds(k, 16)] = plsc.load_gather(
                c2x_v, [lab_v[pl.ds(k, 16)]])

        pltpu.sync_copy(out_v, idxs_hbm)


def _selmerge_body(in_ref, out_ref):
    x = in_ref[...]
    out_ref[...] = jnp.sum(x[:, :SAMP // 128, :], axis=0) - 1


def _run_selmerge(selp):
    return pl.pallas_call(
        _selmerge_body,
        out_shape=jax.ShapeDtypeStruct((SAMP // 128, 128), jnp.int32),
    )(selp.reshape(32, SAMP_PAD // 128, 128))


@functools.lru_cache(maxsize=None)
def _gather_rows_kernel():
    return functools.partial(
        pl.kernel,
        mesh=_vmesh(),
        out_type=jax.ShapeDtypeStruct((SAMP, FDIM), jnp.float32),
        scratch_types=[pltpu.VMEM((SAMP // 32,), jnp.int32),
                       pltpu.VMEM((SAMP // 32, FDIM), jnp.float32),
                       pltpu.SemaphoreType.DMA],
    )(_gather_rows_body)


def _gather_rows_body(sel_hbm, w_hbm, out_hbm, idx_v, rows_v, sem):
    wid = lax.axis_index("s") * 2 + lax.axis_index("c")
    base = wid * (SAMP // 32)
    pltpu.sync_copy(sel_hbm.at[pl.ds(base, SAMP // 32)], idx_v)
    pltpu.async_copy(w_hbm.at[idx_v], rows_v, sem).wait()
    pltpu.sync_copy(rows_v, out_hbm.at[pl.ds(base, SAMP // 32)])


def kernel(features, labels, W):
    scores, gm3 = _run_scores(features, W)
    gm = jnp.transpose(gm3, (1, 0, 2)).reshape(B, ROWS)
    flat, seg = _run_segtop(gm)
    cand = _gather_cand_kernel()(flat[:, :NNBR].reshape(-1),
                                 scores.reshape(-1, 128))
    ids_fast, flag = _run_cand(cand.reshape(B, NCAND), seg, gm)

    def _slow():
        vals, ids = _run_topk(features, W)
        return _run_merge(vals, ids)

    nbr_pad = lax.cond(flag[0, 0] == 0, lambda: ids_fast, _slow)
    nbrs = nbr_pad[:, :NNBR].reshape(-1)
    zer = jnp.zeros((PAD_CLS,), jnp.float32)
    maskA, maskB = _scatter_masks_kernel()(zer, nbrs, labels)
    pos, valid, c2x = _run_positions(maskA, maskB)
    selp, idxs = _compact_and_ranks_kernel()(pos.reshape(-1),
                                             valid.reshape(-1),
                                             c2x.reshape(-1), labels)
    sel = _run_selmerge(selp).reshape(-1)
    weights = _gather_rows_kernel()(sel, W)
    bias = jnp.zeros((SAMP,), jnp.float32)
    return weights, bias, idxs
